# Initial kernel scaffold; baseline (speedup 1.0000x reference)
import jax
import jax.numpy as jnp
from jax.experimental import pallas as pl


def kernel(x, edge_index, Wl1, bl1, Wr1, Wl2, bl2, Wr2, Wlin, blin):
    raise NotImplementedError("write your pallas kernel here")


# speedup vs baseline: 7.0441x; 7.0441x over previous
import functools

import jax
import jax.numpy as jnp
from jax import lax
from jax.experimental import pallas as pl
from jax.experimental.pallas import tpu as pltpu
from jax.experimental.pallas import tpu_sc as plsc

N = 10000
E = 320000
F = 128
W = 128
NWIN = E // W
NPAD = 10240
WIN_PER_SC = NWIN // 2
ROWS_PER_TILE = NPAD // 16
CNT_PER_TILE = NPAD // 16
R = 1000
GRID = N // R

def _sc_edge(x_hbm, srcw_hbm, dstw_hbm, s1_out, cnt_out,
             src_v, dst_v, rows_v, ones_v, zrow_v, zcnt_v,
             acc_sh, cnt_sh, sem):
    cid = lax.axis_index("c")
    sid = lax.axis_index("s")

    for i in range(W // 16):
        ones_v[pl.ds(i * 16, 16)] = jnp.full((16,), 1.0, jnp.float32)
    for i in range(CNT_PER_TILE // 16):
        zcnt_v[pl.ds(i * 16, 16)] = jnp.zeros((16,), jnp.float32)

    def _zrow_fill(r, carry):
        for c8 in range(F // 16):
            zrow_v[r, pl.ds(c8 * 16, 16)] = jnp.zeros((16,), jnp.float32)
        return carry
    lax.fori_loop(0, 32, _zrow_fill, 0)

    row0 = sid * ROWS_PER_TILE

    def _zero_acc(k, carry):
        pltpu.sync_copy(zrow_v, acc_sh.at[pl.ds(row0 + k * 32, 32)])
        return carry
    lax.fori_loop(0, ROWS_PER_TILE // 32, _zero_acc, 0)
    pltpu.sync_copy(zcnt_v, cnt_sh.at[pl.ds(sid * CNT_PER_TILE, CNT_PER_TILE)])
    plsc.subcore_barrier()

    base = cid * WIN_PER_SC
    nwin_t = (WIN_PER_SC - sid + 15) // 16

    def _body(j, carry):
        win = base + sid + j * 16
        pltpu.sync_copy(srcw_hbm.at[pl.ds(win * W, W)], src_v)
        pltpu.sync_copy(dstw_hbm.at[pl.ds(win * W, W)], dst_v)
        pltpu.async_copy(x_hbm.at[src_v], rows_v, sem).wait()
        pltpu.sync_copy(rows_v, acc_sh.at[dst_v], add=True)
        pltpu.sync_copy(ones_v, cnt_sh.at[dst_v], add=True)
        return carry
    lax.fori_loop(0, nwin_t, _body, 0)
    plsc.subcore_barrier()

    pltpu.sync_copy(acc_sh.at[pl.ds(row0, ROWS_PER_TILE)],
                    s1_out.at[pl.ds(cid * NPAD + row0, ROWS_PER_TILE)])
    c0 = sid * CNT_PER_TILE
    pltpu.sync_copy(cnt_sh.at[pl.ds(c0, CNT_PER_TILE)],
                    cnt_out.at[pl.ds(cid * NPAD + c0, CNT_PER_TILE)])


def _sc_scalar(a_hbm, srcw_hbm, dstw_hbm, s2_out,
               src_v, dst_v, vals_v, zcnt_v, s2_sh, sem):
    cid = lax.axis_index("c")
    sid = lax.axis_index("s")

    for i in range(CNT_PER_TILE // 16):
        zcnt_v[pl.ds(i * 16, 16)] = jnp.zeros((16,), jnp.float32)
    c0 = sid * CNT_PER_TILE
    pltpu.sync_copy(zcnt_v, s2_sh.at[pl.ds(c0, CNT_PER_TILE)])
    plsc.subcore_barrier()

    base = cid * WIN_PER_SC
    nwin_t = (WIN_PER_SC - sid + 15) // 16

    def _body(j, carry):
        win = base + sid + j * 16
        pltpu.sync_copy(srcw_hbm.at[pl.ds(win * W, W)], src_v)
        pltpu.sync_copy(dstw_hbm.at[pl.ds(win * W, W)], dst_v)
        pltpu.async_copy(a_hbm.at[src_v], vals_v, sem).wait()
        pltpu.sync_copy(vals_v, s2_sh.at[dst_v], add=True)
        return carry
    lax.fori_loop(0, nwin_t, _body, 0)
    plsc.subcore_barrier()

    pltpu.sync_copy(s2_sh.at[pl.ds(c0, CNT_PER_TILE)],
                    s2_out.at[pl.ds(cid * NPAD + c0, CNT_PER_TILE)])


def _tc1_body(s1a, s1b, cnta, cntb, xb, wl1t, wr1t, wl2t, wr2t, wlin_t, bl1,
              a_out, b_out):
    cnt = jnp.maximum(cnta[...] + cntb[...], 1.0)
    mean = (s1a[...] + s1b[...]) / cnt
    h1 = jnp.maximum(
        jnp.dot(mean, wl1t[...], preferred_element_type=jnp.float32)
        + bl1[...]
        + jnp.dot(xb[...], wr1t[...], preferred_element_type=jnp.float32),
        0.0)
    u_col = jnp.dot(wl2t[...], wlin_t[...], preferred_element_type=jnp.float32)
    w_col = jnp.dot(wr2t[...], wlin_t[...], preferred_element_type=jnp.float32)
    a_out[...] = jnp.dot(h1, u_col, preferred_element_type=jnp.float32)
    b_out[...] = jnp.dot(h1, w_col, preferred_element_type=jnp.float32)


_tc1 = pl.pallas_call(
    _tc1_body,
    grid=(GRID,),
    in_specs=[
        pl.BlockSpec((R, F), lambda i: (i, 0)),
        pl.BlockSpec((R, F), lambda i: (i, 0)),
        pl.BlockSpec((R, 1), lambda i: (i, 0)),
        pl.BlockSpec((R, 1), lambda i: (i, 0)),
        pl.BlockSpec((R, F), lambda i: (i, 0)),
        pl.BlockSpec((F, F), lambda i: (0, 0)),
        pl.BlockSpec((F, F), lambda i: (0, 0)),
        pl.BlockSpec((F, F), lambda i: (0, 0)),
        pl.BlockSpec((F, F), lambda i: (0, 0)),
        pl.BlockSpec((F, 1), lambda i: (0, 0)),
        pl.BlockSpec((1, F), lambda i: (0, 0)),
    ],
    out_specs=[
        pl.BlockSpec((R, 1), lambda i: (i, 0)),
        pl.BlockSpec((R, 1), lambda i: (i, 0)),
    ],
    out_shape=[
        jax.ShapeDtypeStruct((N, 1), jnp.float32),
        jax.ShapeDtypeStruct((N, 1), jnp.float32),
    ],
)


def _tc2_body(s2a, s2b, cnta, cntb, bcol, wlin, bl2_col, blin_c, out):
    cnt = jnp.maximum(cnta[...] + cntb[...], 1.0)
    c = (jnp.dot(wlin[...], bl2_col[...], preferred_element_type=jnp.float32)
         + blin_c[...])
    out[...] = (s2a[...] + s2b[...]) / cnt + bcol[...] + c


_tc2 = pl.pallas_call(
    _tc2_body,
    grid=(GRID,),
    in_specs=[
        pl.BlockSpec((R, 1), lambda i: (i, 0)),
        pl.BlockSpec((R, 1), lambda i: (i, 0)),
        pl.BlockSpec((R, 1), lambda i: (i, 0)),
        pl.BlockSpec((R, 1), lambda i: (i, 0)),
        pl.BlockSpec((R, 1), lambda i: (i, 0)),
        pl.BlockSpec((1, F), lambda i: (0, 0)),
        pl.BlockSpec((F, 1), lambda i: (0, 0)),
        pl.BlockSpec((1, 1), lambda i: (0, 0)),
    ],
    out_specs=pl.BlockSpec((R, 1), lambda i: (i, 0)),
    out_shape=jax.ShapeDtypeStruct((N, 1), jnp.float32),
)


@functools.cache
def _sc_kernels():
    mesh = plsc.VectorSubcoreMesh(core_axis_name="c", subcore_axis_name="s")
    sc_edge = pl.kernel(
        _sc_edge,
        mesh=mesh,
        out_type=[
            jax.ShapeDtypeStruct((2 * NPAD, F), jnp.float32),
            jax.ShapeDtypeStruct((2 * NPAD,), jnp.float32),
        ],
        scratch_types=[
            pltpu.VMEM((W,), jnp.int32),
            pltpu.VMEM((W,), jnp.int32),
            pltpu.VMEM((W, F), jnp.float32),
            pltpu.VMEM((W,), jnp.float32),
            pltpu.VMEM((32, F), jnp.float32),
            pltpu.VMEM((CNT_PER_TILE,), jnp.float32),
            pltpu.VMEM_SHARED((NPAD, F), jnp.float32),
            pltpu.VMEM_SHARED((NPAD,), jnp.float32),
            pltpu.SemaphoreType.DMA,
        ],
    )
    sc_scalar = pl.kernel(
        _sc_scalar,
        mesh=mesh,
        out_type=jax.ShapeDtypeStruct((2 * NPAD,), jnp.float32),
        scratch_types=[
            pltpu.VMEM((W,), jnp.int32),
            pltpu.VMEM((W,), jnp.int32),
            pltpu.VMEM((W,), jnp.float32),
            pltpu.VMEM((CNT_PER_TILE,), jnp.float32),
            pltpu.VMEM_SHARED((NPAD,), jnp.float32),
            pltpu.SemaphoreType.DMA,
        ],
    )
    return sc_edge, sc_scalar


def kernel(x, edge_index, Wl1, bl1, Wr1, Wl2, bl2, Wr2, Wlin, blin):
    sc_edge, sc_scalar = _sc_kernels()
    srcw = edge_index[0]
    dstw = edge_index[1]

    s1_flat, cnt_flat = sc_edge(x, srcw, dstw)
    s1a, s1b = s1_flat[:N], s1_flat[NPAD:NPAD + N]
    cnta = cnt_flat[:N, None]
    cntb = cnt_flat[NPAD:NPAD + N, None]

    a_col, b_col = _tc1(s1a, s1b, cnta, cntb, x,
                        Wl1.T, Wr1.T, Wl2.T, Wr2.T, Wlin.T, bl1[None, :])

    s2_flat = sc_scalar(a_col[:, 0], srcw, dstw)
    s2a = s2_flat[:N, None]
    s2b = s2_flat[NPAD:NPAD + N, None]

    out_col = _tc2(s2a, s2b, cnta, cntb, b_col,
                   Wlin, bl2[:, None], blin[:, None])
    return out_col[:, 0]


# speedup vs baseline: 10.7188x; 1.5217x over previous
import functools

import jax
import jax.numpy as jnp
from jax import lax
from jax.experimental import pallas as pl
from jax.experimental.pallas import tpu as pltpu
from jax.experimental.pallas import tpu_sc as plsc

N = 10000
E = 320000
F = 128
FH = F // 2
NPAD = 10240
EPAD = 327680
W = 64
NWT = EPAD // W // 16
NCHUNK = NWT // 8
ROWS_PER_TILE = NPAD // 16
HALF = NPAD // 2
FIN = HALF // 16
R = 1000
GRID = N // R


def _pipeline(nchunk, fire_idx, drain_idx, fire_g, drain_g, fire_s, drain_s,
              sync_idx0):
    sync_idx0()
    fire_idx(0, 1, 1)
    fire_g(0, 0, 0)

    def _body(t, carry):
        drain_g(0)
        fire_s(0, 0, 0)
        pl.when(t > 0)(lambda: drain_s(1, 1, 1))
        pl.when(t > 0)(lambda: fire_idx(t, 1, 1))
        fire_g(1, 0, 1)
        drain_g(1)
        fire_s(1, 0, 1)
        drain_s(0, 0, 0)
        drain_idx(1)
        fire_g(0, 1, 0)
        drain_g(0)
        fire_s(0, 1, 0)
        drain_s(1, 0, 1)
        pl.when(t < nchunk // 2 - 1)(lambda: fire_idx(t, 2, 0))
        fire_g(1, 1, 1)
        drain_g(1)
        fire_s(1, 1, 1)
        drain_s(0, 1, 0)
        pl.when(t < nchunk // 2 - 1)(lambda: drain_idx(0))
        pl.when(t < nchunk // 2 - 1)(lambda: fire_g(0, 0, 0))
        return carry

    lax.fori_loop(0, nchunk // 2, _body, 0)
    drain_s(1, 1, 1)


def _sc_edge(x2_hbm, src_hbm, dst2_hbm, s1_out, cnt_out,
             sidx_v, didx_v, rows_v, ones_v, zrow_v, zcnt_v,
             acc_sh, cnt_sh, gsem, ssem, isem, zsem):
    cid = lax.axis_index("c")
    sid = lax.axis_index("s")
    e0 = sid * (NWT * W)
    w0 = sid * NWT
    srcoff = cid * N

    for i in range(W // 16):
        ones_v[pl.ds(i * 16, 16)] = jnp.full((16,), 1.0, jnp.float32)
    for i in range(ROWS_PER_TILE // 16):
        zcnt_v[pl.ds(i * 16, 16)] = jnp.zeros((16,), jnp.float32)

    def _zrow_fill(r, carry):
        for c8 in range(FH // 16):
            zrow_v[r, pl.ds(c8 * 16, 16)] = jnp.zeros((16,), jnp.float32)
        return carry
    lax.fori_loop(0, 32, _zrow_fill, 0)

    row0 = sid * ROWS_PER_TILE
    for k in range(ROWS_PER_TILE // 32):
        pltpu.async_copy(zrow_v, acc_sh.at[pl.ds(row0 + k * 32, 32)], zsem)
    pltpu.async_copy(zcnt_v, cnt_sh.at[pl.ds(row0, ROWS_PER_TILE)], zsem)
    for k in range(ROWS_PER_TILE // 32):
        pltpu.make_async_copy(zrow_v, acc_sh.at[pl.ds(row0, 32)], zsem).wait()
    pltpu.make_async_copy(zcnt_v, cnt_sh.at[pl.ds(row0, ROWS_PER_TILE)],
                          zsem).wait()
    plsc.subcore_barrier()

    def _fire_idx(t, chunk_off, slot):
        c = 2 * t + chunk_off
        pltpu.async_copy(src_hbm.at[pl.ds(e0 + c * (8 * W), 8 * W)],
                         sidx_v.at[slot], isem)
        pltpu.async_copy(dst2_hbm.at[pl.ds(w0 + c * 8, 8)],
                         didx_v.at[slot], isem)

    def _remap_src(slot):
        for i in range(8 * W // 16):
            s16 = sidx_v[slot, pl.ds(i * 16, 16)]
            sidx_v[slot, pl.ds(i * 16, 16)] = s16 + srcoff

    def _drain_idx(slot):
        pltpu.make_async_copy(src_hbm.at[pl.ds(0, 8 * W)],
                              sidx_v.at[slot], isem).wait()
        pltpu.make_async_copy(dst2_hbm.at[pl.ds(0, 8)],
                              didx_v.at[slot], isem).wait()
        _remap_src(slot)

    def _fire_g(B, slot, r):
        for k in range(4):
            pltpu.async_copy(
                x2_hbm.at[sidx_v.at[slot, pl.ds((r * 4 + k) * W, W)]],
                rows_v.at[4 * B + k], gsem.at[B])

    def _drain_g(B):
        for k in range(4):
            pltpu.make_async_copy(x2_hbm.at[pl.ds(0, W)],
                                  rows_v.at[4 * B + k], gsem.at[B]).wait()

    def _fire_s(B, slot, r):
        for k in range(4):
            pltpu.async_copy(rows_v.at[4 * B + k],
                             acc_sh.at[didx_v.at[slot, r * 4 + k]],
                             ssem.at[B], add=True)

        def _fire_cnt():
            for k in range(4):
                pltpu.async_copy(ones_v,
                                 cnt_sh.at[didx_v.at[slot, r * 4 + k]],
                                 ssem.at[B], add=True)
        pl.when(cid == 0)(_fire_cnt)

    def _drain_s(B, slot, r):
        for k in range(4):
            pltpu.make_async_copy(rows_v.at[4 * B + k],
                                  acc_sh.at[didx_v.at[slot, r * 4 + k]],
                                  ssem.at[B]).wait()

        def _drain_cnt():
            for k in range(4):
                pltpu.make_async_copy(ones_v,
                                      cnt_sh.at[didx_v.at[slot, r * 4 + k]],
                                      ssem.at[B]).wait()
        pl.when(cid == 0)(_drain_cnt)

    def _sync_idx0():
        pltpu.sync_copy(src_hbm.at[pl.ds(e0, 8 * W)], sidx_v.at[0])
        pltpu.sync_copy(dst2_hbm.at[pl.ds(w0, 8)], didx_v.at[0])
        _remap_src(0)

    _pipeline(NCHUNK, _fire_idx, _drain_idx, _fire_g, _drain_g,
              _fire_s, _drain_s, _sync_idx0)
    plsc.subcore_barrier()

    pltpu.sync_copy(acc_sh.at[pl.ds(row0, ROWS_PER_TILE)],
                    s1_out.at[pl.ds(cid * NPAD + row0, ROWS_PER_TILE)])
    pl.when(cid == 0)(lambda: pltpu.sync_copy(
        cnt_sh.at[pl.ds(row0, ROWS_PER_TILE)],
        cnt_out.at[pl.ds(row0, ROWS_PER_TILE)]))


def _sc_scalar(a_hbm, src_hbm, dst2_hbm, cnt_hbm, bc_hbm, out_hbm,
               sidx_v, didx_v, vals_v, zcnt_v, fs2_v, fcnt_v, fbc_v, fout_v,
               s2_sh, gsem, ssem, isem):
    cid = lax.axis_index("c")
    sid = lax.axis_index("s")
    e0 = sid * (NWT * W)
    w0 = sid * NWT
    lo = cid * HALF

    for i in range(ROWS_PER_TILE // 16):
        zcnt_v[pl.ds(i * 16, 16)] = jnp.zeros((16,), jnp.float32)
    row0 = sid * ROWS_PER_TILE
    pltpu.sync_copy(zcnt_v, s2_sh.at[pl.ds(row0, ROWS_PER_TILE)])
    plsc.subcore_barrier()

    def _fire_idx(t, chunk_off, slot):
        c = 2 * t + chunk_off
        pltpu.async_copy(src_hbm.at[pl.ds(e0 + c * (8 * W), 8 * W)],
                         sidx_v.at[slot], isem)
        pltpu.async_copy(dst2_hbm.at[pl.ds(w0 + c * 8, 8)],
                         didx_v.at[slot], isem)

    def _remap_dst(slot):
        for k in range(8):
            for i in range(W // 16):
                d16 = didx_v[slot, k, pl.ds(i * 16, 16)]
                keep = (d16 >= lo) & (d16 < lo + HALF)
                dump = (d16 & 127) + (N * (1 - cid))
                didx_v[slot, k, pl.ds(i * 16, 16)] = jnp.where(keep, d16, dump)

    def _drain_idx(slot):
        pltpu.make_async_copy(src_hbm.at[pl.ds(0, 8 * W)],
                              sidx_v.at[slot], isem).wait()
        pltpu.make_async_copy(dst2_hbm.at[pl.ds(0, 8)],
                              didx_v.at[slot], isem).wait()
        _remap_dst(slot)

    def _fire_g(B, slot, r):
        for k in range(4):
            pltpu.async_copy(
                a_hbm.at[sidx_v.at[slot, pl.ds((r * 4 + k) * W, W)]],
                vals_v.at[4 * B + k], gsem.at[B])

    def _drain_g(B):
        for k in range(4):
            pltpu.make_async_copy(a_hbm.at[pl.ds(0, W)],
                                  vals_v.at[4 * B + k], gsem.at[B]).wait()

    def _fire_s(B, slot, r):
        for k in range(4):
            pltpu.async_copy(vals_v.at[4 * B + k],
                             s2_sh.at[didx_v.at[slot, r * 4 + k]],
                             ssem.at[B], add=True)

    def _drain_s(B, slot, r):
        for k in range(4):
            pltpu.make_async_copy(vals_v.at[4 * B + k],
                                  s2_sh.at[didx_v.at[slot, r * 4 + k]],
                                  ssem.at[B]).wait()

    def _sync_idx0():
        pltpu.sync_copy(src_hbm.at[pl.ds(e0, 8 * W)], sidx_v.at[0])
        pltpu.sync_copy(dst2_hbm.at[pl.ds(w0, 8)], didx_v.at[0])
        _remap_dst(0)

    _pipeline(NCHUNK, _fire_idx, _drain_idx, _fire_g, _drain_g,
              _fire_s, _drain_s, _sync_idx0)
    plsc.subcore_barrier()

    rowf = lo + sid * FIN
    pltpu.sync_copy(s2_sh.at[pl.ds(rowf, FIN)], fs2_v)
    pltpu.sync_copy(cnt_hbm.at[pl.ds(rowf, FIN)], fcnt_v)
    pltpu.sync_copy(bc_hbm.at[pl.ds(rowf, FIN)], fbc_v)
    for i in range(FIN // 16):
        s2v = fs2_v[pl.ds(i * 16, 16)]
        cntv = jnp.maximum(fcnt_v[pl.ds(i * 16, 16)], 1.0)
        bcv = fbc_v[pl.ds(i * 16, 16)]
        fout_v[pl.ds(i * 16, 16)] = s2v / cntv + bcv
    pltpu.sync_copy(fout_v, out_hbm.at[pl.ds(rowf, FIN)])


@functools.cache
def _sc_kernels():
    mesh = plsc.VectorSubcoreMesh(core_axis_name="c", subcore_axis_name="s")
    sc_edge = pl.kernel(
        _sc_edge,
        mesh=mesh,
        compiler_params=pltpu.CompilerParams(use_tc_tiling_on_sc=False),
        out_type=[
            jax.ShapeDtypeStruct((2 * NPAD, FH), jnp.float32),
            jax.ShapeDtypeStruct((NPAD,), jnp.float32),
        ],
        scratch_types=[
            pltpu.VMEM((2, 8 * W), jnp.int32),
            pltpu.VMEM((2, 8, W), jnp.int32),
            pltpu.VMEM((8, W, FH), jnp.float32),
            pltpu.VMEM((W,), jnp.float32),
            pltpu.VMEM((32, FH), jnp.float32),
            pltpu.VMEM((ROWS_PER_TILE,), jnp.float32),
            pltpu.VMEM_SHARED((NPAD, FH), jnp.float32),
            pltpu.VMEM_SHARED((NPAD,), jnp.float32),
            pltpu.SemaphoreType.DMA((2,)),
            pltpu.SemaphoreType.DMA((2,)),
            pltpu.SemaphoreType.DMA,
            pltpu.SemaphoreType.DMA,
        ],
    )
    sc_scalar = pl.kernel(
        _sc_scalar,
        mesh=mesh,
        out_type=jax.ShapeDtypeStruct((NPAD,), jnp.float32),
        scratch_types=[
            pltpu.VMEM((2, 8 * W), jnp.int32),
            pltpu.VMEM((2, 8, W), jnp.int32),
            pltpu.VMEM((8, W), jnp.float32),
            pltpu.VMEM((ROWS_PER_TILE,), jnp.float32),
            pltpu.VMEM((FIN,), jnp.float32),
            pltpu.VMEM((FIN,), jnp.float32),
            pltpu.VMEM((FIN,), jnp.float32),
            pltpu.VMEM((FIN,), jnp.float32),
            pltpu.VMEM_SHARED((NPAD,), jnp.float32),
            pltpu.SemaphoreType.DMA((2,)),
            pltpu.SemaphoreType.DMA((2,)),
            pltpu.SemaphoreType.DMA,
        ],
    )
    return sc_edge, sc_scalar


def _tc1_body(s1a, s1b, cnt, xb, wl1ta, wl1tb, wr1t, wl2t, wr2t, wlin_t,
              bl1, bl2_col, blin_c, a_out, bc_out):
    cnt_c = jnp.maximum(cnt[...], 1.0)
    h1 = jnp.maximum(
        jnp.dot(s1a[...] / cnt_c, wl1ta[...],
                preferred_element_type=jnp.float32)
        + jnp.dot(s1b[...] / cnt_c, wl1tb[...],
                  preferred_element_type=jnp.float32)
        + bl1[...]
        + jnp.dot(xb[...], wr1t[...], preferred_element_type=jnp.float32),
        0.0)
    u_col = jnp.dot(wl2t[...], wlin_t[...], preferred_element_type=jnp.float32)
    w_col = jnp.dot(wr2t[...], wlin_t[...], preferred_element_type=jnp.float32)
    c = (jnp.dot(bl2_col[...].T, wlin_t[...],
                 preferred_element_type=jnp.float32) + blin_c[...])
    a_out[...] = jnp.dot(h1, u_col, preferred_element_type=jnp.float32)
    bc_out[...] = jnp.dot(h1, w_col, preferred_element_type=jnp.float32) + c


_tc1 = pl.pallas_call(
    _tc1_body,
    grid=(GRID,),
    in_specs=[
        pl.BlockSpec((R, FH), lambda i: (i, 0)),
        pl.BlockSpec((R, FH), lambda i: (i, 0)),
        pl.BlockSpec((R, 1), lambda i: (i, 0)),
        pl.BlockSpec((R, F), lambda i: (i, 0)),
        pl.BlockSpec((FH, F), lambda i: (0, 0)),
        pl.BlockSpec((FH, F), lambda i: (0, 0)),
        pl.BlockSpec((F, F), lambda i: (0, 0)),
        pl.BlockSpec((F, F), lambda i: (0, 0)),
        pl.BlockSpec((F, F), lambda i: (0, 0)),
        pl.BlockSpec((F, 1), lambda i: (0, 0)),
        pl.BlockSpec((1, F), lambda i: (0, 0)),
        pl.BlockSpec((F, 1), lambda i: (0, 0)),
        pl.BlockSpec((1, 1), lambda i: (0, 0)),
    ],
    out_specs=[
        pl.BlockSpec((R, 1), lambda i: (i, 0)),
        pl.BlockSpec((R, 1), lambda i: (i, 0)),
    ],
    out_shape=[
        jax.ShapeDtypeStruct((N, 1), jnp.float32),
        jax.ShapeDtypeStruct((N, 1), jnp.float32),
    ],
)


def kernel(x, edge_index, Wl1, bl1, Wr1, Wl2, bl2, Wr2, Wlin, blin):
    sc_edge, sc_scalar = _sc_kernels()

    npad_e = EPAD - E
    pad_src = (jnp.arange(npad_e, dtype=jnp.int32) * 37) % N
    pad_dst = N + (jnp.arange(npad_e, dtype=jnp.int32) % (NPAD - N))
    srcp = jnp.concatenate([edge_index[0], pad_src])
    dstp = jnp.concatenate([edge_index[1], pad_dst])
    dst2 = dstp.reshape(EPAD // W, W)
    x2 = jnp.concatenate([x[:, :FH], x[:, FH:]], axis=0)

    s1_flat, cnt_flat = sc_edge(x2, srcp, dst2)
    s1a, s1b = s1_flat[:N], s1_flat[NPAD:NPAD + N]
    cnt_col = cnt_flat[:N, None]

    a_col, bc_col = _tc1(s1a, s1b, cnt_col, x,
                         Wl1.T[:FH], Wl1.T[FH:], Wr1.T, Wl2.T, Wr2.T, Wlin.T,
                         bl1[None, :], bl2[:, None], blin[:, None])
    a_pad = jnp.pad(a_col[:, 0], (0, NPAD - N))
    bc_pad = jnp.pad(bc_col[:, 0], (0, NPAD - N))

    out_pad = sc_scalar(a_pad, srcp, dst2, cnt_flat, bc_pad)
    return out_pad[:N]


# speedup vs baseline: 11.8704x; 1.1074x over previous
import functools

import jax
import jax.numpy as jnp
from jax import lax
from jax.experimental import pallas as pl
from jax.experimental.pallas import tpu as pltpu
from jax.experimental.pallas import tpu_sc as plsc

N = 10000
E = 320000
F = 128
FH = F // 2
NPAD = 10240
EPAD = 327680
W = 128
NWT = EPAD // W // 16
NCHUNK = NWT // 8
ROWS_PER_TILE = NPAD // 16
HALF = NPAD // 2
FIN = HALF // 16
R = 1000
GRID = N // R


def _pipeline(nchunk, fire_idx, drain_idx, fire_g, drain_g, fire_s, drain_s,
              sync_idx0):
    sync_idx0()
    fire_idx(0, 1, 1)
    fire_g(0, 0, 0)

    def _body(t, carry):
        drain_g(0)
        fire_s(0, 0, 0)
        pl.when(t > 0)(lambda: drain_s(1, 1, 1))
        pl.when(t > 0)(lambda: fire_idx(t, 1, 1))
        fire_g(1, 0, 1)
        drain_g(1)
        fire_s(1, 0, 1)
        drain_s(0, 0, 0)
        drain_idx(1)
        fire_g(0, 1, 0)
        drain_g(0)
        fire_s(0, 1, 0)
        drain_s(1, 0, 1)
        pl.when(t < nchunk // 2 - 1)(lambda: fire_idx(t, 2, 0))
        fire_g(1, 1, 1)
        drain_g(1)
        fire_s(1, 1, 1)
        drain_s(0, 1, 0)
        pl.when(t < nchunk // 2 - 1)(lambda: drain_idx(0))
        pl.when(t < nchunk // 2 - 1)(lambda: fire_g(0, 0, 0))
        return carry

    lax.fori_loop(0, nchunk // 2, _body, 0)
    drain_s(1, 1, 1)


def _sc_edge(x2_hbm, src_hbm, dst2_hbm, s1_out, cnt_out,
             sidx_v, didx_v, rows_v, ones_v, zrow_v, zcnt_v,
             acc_sh, cnt_sh, gsem, ssem, isem, zsem):
    cid = lax.axis_index("c")
    sid = lax.axis_index("s")
    e0 = sid * (NWT * W)
    w0 = sid * NWT
    srcoff = cid * N

    for i in range(W // 16):
        ones_v[pl.ds(i * 16, 16)] = jnp.full((16,), 1.0, jnp.float32)
    for i in range(ROWS_PER_TILE // 16):
        zcnt_v[pl.ds(i * 16, 16)] = jnp.zeros((16,), jnp.float32)

    def _zrow_fill(r, carry):
        for c8 in range(FH // 16):
            zrow_v[r, pl.ds(c8 * 16, 16)] = jnp.zeros((16,), jnp.float32)
        return carry
    lax.fori_loop(0, 32, _zrow_fill, 0)

    row0 = sid * ROWS_PER_TILE
    for k in range(ROWS_PER_TILE // 32):
        pltpu.async_copy(zrow_v, acc_sh.at[pl.ds(row0 + k * 32, 32)], zsem)
    pltpu.async_copy(zcnt_v, cnt_sh.at[pl.ds(row0, ROWS_PER_TILE)], zsem)
    for k in range(ROWS_PER_TILE // 32):
        pltpu.make_async_copy(zrow_v, acc_sh.at[pl.ds(row0, 32)], zsem).wait()
    pltpu.make_async_copy(zcnt_v, cnt_sh.at[pl.ds(row0, ROWS_PER_TILE)],
                          zsem).wait()
    plsc.subcore_barrier()

    def _fire_idx(t, chunk_off, slot):
        c = 2 * t + chunk_off
        pltpu.async_copy(src_hbm.at[pl.ds(e0 + c * (8 * W), 8 * W)],
                         sidx_v.at[slot], isem)
        pltpu.async_copy(dst2_hbm.at[pl.ds(w0 + c * 8, 8)],
                         didx_v.at[slot], isem)

    def _remap_src(slot):
        for i in range(8 * W // 16):
            s16 = sidx_v[slot, pl.ds(i * 16, 16)]
            sidx_v[slot, pl.ds(i * 16, 16)] = s16 + srcoff

    def _drain_idx(slot):
        pltpu.make_async_copy(src_hbm.at[pl.ds(0, 8 * W)],
                              sidx_v.at[slot], isem).wait()
        pltpu.make_async_copy(dst2_hbm.at[pl.ds(0, 8)],
                              didx_v.at[slot], isem).wait()
        _remap_src(slot)

    def _fire_g(B, slot, r):
        for k in range(4):
            pltpu.async_copy(
                x2_hbm.at[sidx_v.at[slot, pl.ds((r * 4 + k) * W, W)]],
                rows_v.at[4 * B + k], gsem.at[B])

    def _drain_g(B):
        for k in range(4):
            pltpu.make_async_copy(x2_hbm.at[pl.ds(0, W)],
                                  rows_v.at[4 * B + k], gsem.at[B]).wait()

    def _fire_s(B, slot, r):
        for k in range(4):
            pltpu.async_copy(rows_v.at[4 * B + k],
                             acc_sh.at[didx_v.at[slot, r * 4 + k]],
                             ssem.at[B], add=True)

        def _fire_cnt():
            for k in range(4):
                pltpu.async_copy(ones_v,
                                 cnt_sh.at[didx_v.at[slot, r * 4 + k]],
                                 ssem.at[B], add=True)
        pl.when(cid == 0)(_fire_cnt)

    def _drain_s(B, slot, r):
        for k in range(4):
            pltpu.make_async_copy(rows_v.at[4 * B + k],
                                  acc_sh.at[didx_v.at[slot, r * 4 + k]],
                                  ssem.at[B]).wait()

        def _drain_cnt():
            for k in range(4):
                pltpu.make_async_copy(ones_v,
                                      cnt_sh.at[didx_v.at[slot, r * 4 + k]],
                                      ssem.at[B]).wait()
        pl.when(cid == 0)(_drain_cnt)

    def _sync_idx0():
        pltpu.sync_copy(src_hbm.at[pl.ds(e0, 8 * W)], sidx_v.at[0])
        pltpu.sync_copy(dst2_hbm.at[pl.ds(w0, 8)], didx_v.at[0])
        _remap_src(0)

    _pipeline(NCHUNK, _fire_idx, _drain_idx, _fire_g, _drain_g,
              _fire_s, _drain_s, _sync_idx0)
    plsc.subcore_barrier()

    pltpu.sync_copy(acc_sh.at[pl.ds(row0, ROWS_PER_TILE)],
                    s1_out.at[pl.ds(cid * NPAD + row0, ROWS_PER_TILE)])
    pl.when(cid == 0)(lambda: pltpu.sync_copy(
        cnt_sh.at[pl.ds(row0, ROWS_PER_TILE)],
        cnt_out.at[pl.ds(row0, ROWS_PER_TILE)]))


def _sc_scalar(a_hbm, src_hbm, dst2_hbm, cnt_hbm, bc_hbm, out_hbm,
               sidx_v, didx_v, vals_v, zcnt_v, fs2_v, fcnt_v, fbc_v, fout_v,
               s2_sh, gsem, ssem, isem):
    cid = lax.axis_index("c")
    sid = lax.axis_index("s")
    e0 = sid * (NWT * W)
    w0 = sid * NWT
    lo = cid * HALF

    for i in range(ROWS_PER_TILE // 16):
        zcnt_v[pl.ds(i * 16, 16)] = jnp.zeros((16,), jnp.float32)
    row0 = sid * ROWS_PER_TILE
    pltpu.sync_copy(zcnt_v, s2_sh.at[pl.ds(row0, ROWS_PER_TILE)])
    plsc.subcore_barrier()

    def _fire_idx(t, chunk_off, slot):
        c = 2 * t + chunk_off
        pltpu.async_copy(src_hbm.at[pl.ds(e0 + c * (8 * W), 8 * W)],
                         sidx_v.at[slot], isem)
        pltpu.async_copy(dst2_hbm.at[pl.ds(w0 + c * 8, 8)],
                         didx_v.at[slot], isem)

    def _remap_dst(slot):
        for k in range(8):
            for i in range(W // 16):
                d16 = didx_v[slot, k, pl.ds(i * 16, 16)]
                keep = (d16 >= lo) & (d16 < lo + HALF)
                dump = (d16 & 127) + (N * (1 - cid))
                didx_v[slot, k, pl.ds(i * 16, 16)] = jnp.where(keep, d16, dump)

    def _drain_idx(slot):
        pltpu.make_async_copy(src_hbm.at[pl.ds(0, 8 * W)],
                              sidx_v.at[slot], isem).wait()
        pltpu.make_async_copy(dst2_hbm.at[pl.ds(0, 8)],
                              didx_v.at[slot], isem).wait()
        _remap_dst(slot)

    def _fire_g(B, slot, r):
        for k in range(4):
            pltpu.async_copy(
                a_hbm.at[sidx_v.at[slot, pl.ds((r * 4 + k) * W, W)]],
                vals_v.at[4 * B + k], gsem.at[B])

    def _drain_g(B):
        for k in range(4):
            pltpu.make_async_copy(a_hbm.at[pl.ds(0, W)],
                                  vals_v.at[4 * B + k], gsem.at[B]).wait()

    def _fire_s(B, slot, r):
        for k in range(4):
            pltpu.async_copy(vals_v.at[4 * B + k],
                             s2_sh.at[didx_v.at[slot, r * 4 + k]],
                             ssem.at[B], add=True)

    def _drain_s(B, slot, r):
        for k in range(4):
            pltpu.make_async_copy(vals_v.at[4 * B + k],
                                  s2_sh.at[didx_v.at[slot, r * 4 + k]],
                                  ssem.at[B]).wait()

    def _sync_idx0():
        pltpu.sync_copy(src_hbm.at[pl.ds(e0, 8 * W)], sidx_v.at[0])
        pltpu.sync_copy(dst2_hbm.at[pl.ds(w0, 8)], didx_v.at[0])
        _remap_dst(0)

    _pipeline(NCHUNK, _fire_idx, _drain_idx, _fire_g, _drain_g,
              _fire_s, _drain_s, _sync_idx0)
    plsc.subcore_barrier()

    rowf = lo + sid * FIN
    pltpu.sync_copy(s2_sh.at[pl.ds(rowf, FIN)], fs2_v)
    pltpu.sync_copy(cnt_hbm.at[pl.ds(rowf, FIN)], fcnt_v)
    pltpu.sync_copy(bc_hbm.at[pl.ds(rowf, FIN)], fbc_v)
    for i in range(FIN // 16):
        s2v = fs2_v[pl.ds(i * 16, 16)]
        cntv = jnp.maximum(fcnt_v[pl.ds(i * 16, 16)], 1.0)
        bcv = fbc_v[pl.ds(i * 16, 16)]
        fout_v[pl.ds(i * 16, 16)] = s2v / cntv + bcv
    pltpu.sync_copy(fout_v, out_hbm.at[pl.ds(rowf, FIN)])


@functools.cache
def _sc_kernels():
    mesh = plsc.VectorSubcoreMesh(core_axis_name="c", subcore_axis_name="s")
    sc_edge = pl.kernel(
        _sc_edge,
        mesh=mesh,
        compiler_params=pltpu.CompilerParams(use_tc_tiling_on_sc=False),
        out_type=[
            jax.ShapeDtypeStruct((2 * NPAD, FH), jnp.float32),
            jax.ShapeDtypeStruct((NPAD,), jnp.float32),
        ],
        scratch_types=[
            pltpu.VMEM((2, 8 * W), jnp.int32),
            pltpu.VMEM((2, 8, W), jnp.int32),
            pltpu.VMEM((8, W, FH), jnp.float32),
            pltpu.VMEM((W,), jnp.float32),
            pltpu.VMEM((32, FH), jnp.float32),
            pltpu.VMEM((ROWS_PER_TILE,), jnp.float32),
            pltpu.VMEM_SHARED((NPAD, FH), jnp.float32),
            pltpu.VMEM_SHARED((NPAD,), jnp.float32),
            pltpu.SemaphoreType.DMA((2,)),
            pltpu.SemaphoreType.DMA((2,)),
            pltpu.SemaphoreType.DMA,
            pltpu.SemaphoreType.DMA,
        ],
    )
    sc_scalar = pl.kernel(
        _sc_scalar,
        mesh=mesh,
        out_type=jax.ShapeDtypeStruct((NPAD,), jnp.float32),
        scratch_types=[
            pltpu.VMEM((2, 8 * W), jnp.int32),
            pltpu.VMEM((2, 8, W), jnp.int32),
            pltpu.VMEM((8, W), jnp.float32),
            pltpu.VMEM((ROWS_PER_TILE,), jnp.float32),
            pltpu.VMEM((FIN,), jnp.float32),
            pltpu.VMEM((FIN,), jnp.float32),
            pltpu.VMEM((FIN,), jnp.float32),
            pltpu.VMEM((FIN,), jnp.float32),
            pltpu.VMEM_SHARED((NPAD,), jnp.float32),
            pltpu.SemaphoreType.DMA((2,)),
            pltpu.SemaphoreType.DMA((2,)),
            pltpu.SemaphoreType.DMA,
        ],
    )
    return sc_edge, sc_scalar


def _tc1_body(s1a, s1b, cnt, xb, wl1ta, wl1tb, wr1t, wl2t, wr2t, wlin_t,
              bl1, bl2_col, blin_c, a_out, bc_out):
    cnt_c = jnp.maximum(cnt[...], 1.0)
    h1 = jnp.maximum(
        jnp.dot(s1a[...] / cnt_c, wl1ta[...],
                preferred_element_type=jnp.float32)
        + jnp.dot(s1b[...] / cnt_c, wl1tb[...],
                  preferred_element_type=jnp.float32)
        + bl1[...]
        + jnp.dot(xb[...], wr1t[...], preferred_element_type=jnp.float32),
        0.0)
    u_col = jnp.dot(wl2t[...], wlin_t[...], preferred_element_type=jnp.float32)
    w_col = jnp.dot(wr2t[...], wlin_t[...], preferred_element_type=jnp.float32)
    c = (jnp.dot(bl2_col[...].T, wlin_t[...],
                 preferred_element_type=jnp.float32) + blin_c[...])
    a_out[...] = jnp.dot(h1, u_col, preferred_element_type=jnp.float32)
    bc_out[...] = jnp.dot(h1, w_col, preferred_element_type=jnp.float32) + c


_tc1 = pl.pallas_call(
    _tc1_body,
    grid=(GRID,),
    in_specs=[
        pl.BlockSpec((R, FH), lambda i: (i, 0)),
        pl.BlockSpec((R, FH), lambda i: (i, 0)),
        pl.BlockSpec((R, 1), lambda i: (i, 0)),
        pl.BlockSpec((R, F), lambda i: (i, 0)),
        pl.BlockSpec((FH, F), lambda i: (0, 0)),
        pl.BlockSpec((FH, F), lambda i: (0, 0)),
        pl.BlockSpec((F, F), lambda i: (0, 0)),
        pl.BlockSpec((F, F), lambda i: (0, 0)),
        pl.BlockSpec((F, F), lambda i: (0, 0)),
        pl.BlockSpec((F, 1), lambda i: (0, 0)),
        pl.BlockSpec((1, F), lambda i: (0, 0)),
        pl.BlockSpec((F, 1), lambda i: (0, 0)),
        pl.BlockSpec((1, 1), lambda i: (0, 0)),
    ],
    out_specs=[
        pl.BlockSpec((R, 1), lambda i: (i, 0)),
        pl.BlockSpec((R, 1), lambda i: (i, 0)),
    ],
    out_shape=[
        jax.ShapeDtypeStruct((N, 1), jnp.float32),
        jax.ShapeDtypeStruct((N, 1), jnp.float32),
    ],
)


def kernel(x, edge_index, Wl1, bl1, Wr1, Wl2, bl2, Wr2, Wlin, blin):
    sc_edge, sc_scalar = _sc_kernels()

    npad_e = EPAD - E
    pad_src = (jnp.arange(npad_e, dtype=jnp.int32) * 37) % N
    pad_dst = N + (jnp.arange(npad_e, dtype=jnp.int32) % (NPAD - N))
    srcp = jnp.concatenate([edge_index[0], pad_src])
    dstp = jnp.concatenate([edge_index[1], pad_dst])
    dst2 = dstp.reshape(EPAD // W, W)
    x2 = jnp.concatenate([x[:, :FH], x[:, FH:]], axis=0)

    s1_flat, cnt_flat = sc_edge(x2, srcp, dst2)
    s1a, s1b = s1_flat[:N], s1_flat[NPAD:NPAD + N]
    cnt_col = cnt_flat[:N, None]

    a_col, bc_col = _tc1(s1a, s1b, cnt_col, x,
                         Wl1.T[:FH], Wl1.T[FH:], Wr1.T, Wl2.T, Wr2.T, Wlin.T,
                         bl1[None, :], bl2[:, None], blin[:, None])
    a_pad = jnp.pad(a_col[:, 0], (0, NPAD - N))
    bc_pad = jnp.pad(bc_col[:, 0], (0, NPAD - N))

    out_pad = sc_scalar(a_pad, srcp, dst2, cnt_flat, bc_pad)
    return out_pad[:N]


# speedup vs baseline: 14.6975x; 1.2382x over previous
import functools

import jax
import jax.numpy as jnp
from jax import lax
from jax.experimental import pallas as pl
from jax.experimental.pallas import tpu as pltpu
from jax.experimental.pallas import tpu_sc as plsc

N = 10000
E = 320000
F = 128
FH = F // 2
NPAD = 10240
EPAD = 327680
W = 128
NWT = EPAD // W // 16
NCHUNK = NWT // 8
ROWS_PER_TILE = NPAD // 16
HALF = NPAD // 2
FIN = HALF // 16
R = 1000
GRID = N // R


def _pipeline(nchunk, fire_idx, drain_idx, fire_g, drain_g, fire_s, drain_s,
              sync_idx0):
    sync_idx0()
    fire_idx(0, 1, 1)
    fire_g(0, 0, 0)

    def _body(t, carry):
        drain_g(0)
        fire_s(0, 0, 0)
        pl.when(t > 0)(lambda: drain_s(1, 1, 1))
        pl.when(t > 0)(lambda: fire_idx(t, 1, 1))
        fire_g(1, 0, 1)
        drain_g(1)
        fire_s(1, 0, 1)
        drain_s(0, 0, 0)
        drain_idx(1)
        fire_g(0, 1, 0)
        drain_g(0)
        fire_s(0, 1, 0)
        drain_s(1, 0, 1)
        pl.when(t < nchunk // 2 - 1)(lambda: fire_idx(t, 2, 0))
        fire_g(1, 1, 1)
        drain_g(1)
        fire_s(1, 1, 1)
        drain_s(0, 1, 0)
        pl.when(t < nchunk // 2 - 1)(lambda: drain_idx(0))
        pl.when(t < nchunk // 2 - 1)(lambda: fire_g(0, 0, 0))
        return carry

    lax.fori_loop(0, nchunk // 2, _body, 0)
    drain_s(1, 1, 1)


def _sc_edge(x2_hbm, src_hbm, dst2_hbm, s1_out, cnt_out,
             sidx_v, didx_v, rows_v, ones_v, zrow_v, zcnt_v,
             acc_sh, cnt_sh, gsem, ssem, isem, zsem):
    cid = lax.axis_index("c")
    sid = lax.axis_index("s")
    e0 = sid * (NWT * W)
    w0 = sid * NWT
    srcoff = cid * N

    for i in range(W // 16):
        ones_v[pl.ds(i * 16, 16)] = jnp.full((16,), 1.0, jnp.float32)
    for i in range(ROWS_PER_TILE // 16):
        zcnt_v[pl.ds(i * 16, 16)] = jnp.zeros((16,), jnp.float32)

    def _zrow_fill(r, carry):
        for c8 in range(FH // 16):
            zrow_v[r, pl.ds(c8 * 16, 16)] = jnp.zeros((16,), jnp.float32)
        return carry
    lax.fori_loop(0, 32, _zrow_fill, 0)

    row0 = sid * ROWS_PER_TILE
    for k in range(ROWS_PER_TILE // 32):
        pltpu.async_copy(zrow_v, acc_sh.at[pl.ds(row0 + k * 32, 32)], zsem)
    pltpu.async_copy(zcnt_v, cnt_sh.at[pl.ds(row0, ROWS_PER_TILE)], zsem)
    for k in range(ROWS_PER_TILE // 32):
        pltpu.make_async_copy(zrow_v, acc_sh.at[pl.ds(row0, 32)], zsem).wait()
    pltpu.make_async_copy(zcnt_v, cnt_sh.at[pl.ds(row0, ROWS_PER_TILE)],
                          zsem).wait()
    plsc.subcore_barrier()

    def _fire_idx(t, chunk_off, slot):
        c = 2 * t + chunk_off
        pltpu.async_copy(src_hbm.at[pl.ds(e0 + c * (8 * W), 8 * W)],
                         sidx_v.at[slot], isem)
        pltpu.async_copy(dst2_hbm.at[pl.ds(w0 + c * 8, 8)],
                         didx_v.at[slot], isem)

    def _remap_src(slot):
        for i in range(8 * W // 16):
            s16 = sidx_v[slot, pl.ds(i * 16, 16)]
            sidx_v[slot, pl.ds(i * 16, 16)] = s16 + srcoff

    def _drain_idx(slot):
        pltpu.make_async_copy(src_hbm.at[pl.ds(0, 8 * W)],
                              sidx_v.at[slot], isem).wait()
        pltpu.make_async_copy(dst2_hbm.at[pl.ds(0, 8)],
                              didx_v.at[slot], isem).wait()
        _remap_src(slot)

    def _fire_g(B, slot, r):
        for k in range(4):
            pltpu.async_copy(
                x2_hbm.at[sidx_v.at[slot, pl.ds((r * 4 + k) * W, W)]],
                rows_v.at[4 * B + k], gsem.at[B])

    def _drain_g(B):
        for k in range(4):
            pltpu.make_async_copy(x2_hbm.at[pl.ds(0, W)],
                                  rows_v.at[4 * B + k], gsem.at[B]).wait()

    def _fire_s(B, slot, r):
        for k in range(4):
            pltpu.async_copy(rows_v.at[4 * B + k],
                             acc_sh.at[didx_v.at[slot, r * 4 + k]],
                             ssem.at[B], add=True)

        def _fire_cnt():
            for k in range(4):
                pltpu.async_copy(ones_v,
                                 cnt_sh.at[didx_v.at[slot, r * 4 + k]],
                                 ssem.at[B], add=True)
        pl.when(cid == 0)(_fire_cnt)

    def _drain_s(B, slot, r):
        for k in range(4):
            pltpu.make_async_copy(rows_v.at[4 * B + k],
                                  acc_sh.at[didx_v.at[slot, r * 4 + k]],
                                  ssem.at[B]).wait()

        def _drain_cnt():
            for k in range(4):
                pltpu.make_async_copy(ones_v,
                                      cnt_sh.at[didx_v.at[slot, r * 4 + k]],
                                      ssem.at[B]).wait()
        pl.when(cid == 0)(_drain_cnt)

    def _sync_idx0():
        pltpu.sync_copy(src_hbm.at[pl.ds(e0, 8 * W)], sidx_v.at[0])
        pltpu.sync_copy(dst2_hbm.at[pl.ds(w0, 8)], didx_v.at[0])
        _remap_src(0)

    _pipeline(NCHUNK, _fire_idx, _drain_idx, _fire_g, _drain_g,
              _fire_s, _drain_s, _sync_idx0)
    plsc.subcore_barrier()

    pltpu.sync_copy(acc_sh.at[pl.ds(row0, ROWS_PER_TILE)],
                    s1_out.at[pl.ds(cid * NPAD + row0, ROWS_PER_TILE)])
    pl.when(cid == 0)(lambda: pltpu.sync_copy(
        cnt_sh.at[pl.ds(row0, ROWS_PER_TILE)],
        cnt_out.at[pl.ds(row0, ROWS_PER_TILE)]))


def _sc_scalar(a_hbm, src_hbm, dst2_hbm, cnt_hbm, bc_hbm, out_hbm,
               sidx_v, didx_v, vals_v, zcnt_v, fs2_v, fcnt_v, fbc_v, fout_v,
               s2_sh, a_sh, gsem, ssem, isem):
    cid = lax.axis_index("c")
    sid = lax.axis_index("s")
    e0 = sid * (NWT * W)
    w0 = sid * NWT
    lo = cid * HALF

    for i in range(ROWS_PER_TILE // 16):
        zcnt_v[pl.ds(i * 16, 16)] = jnp.zeros((16,), jnp.float32)
    row0 = sid * ROWS_PER_TILE
    pltpu.sync_copy(zcnt_v, s2_sh.at[pl.ds(row0, ROWS_PER_TILE)])
    pl.when(sid == 0)(lambda: pltpu.sync_copy(a_hbm, a_sh))
    plsc.subcore_barrier()

    def _fire_idx(t, chunk_off, slot):
        c = 2 * t + chunk_off
        pltpu.async_copy(src_hbm.at[pl.ds(e0 + c * (8 * W), 8 * W)],
                         sidx_v.at[slot], isem)
        pltpu.async_copy(dst2_hbm.at[pl.ds(w0 + c * 8, 8)],
                         didx_v.at[slot], isem)

    def _remap_dst(slot):
        for k in range(8):
            for i in range(W // 16):
                d16 = didx_v[slot, k, pl.ds(i * 16, 16)]
                keep = (d16 >= lo) & (d16 < lo + HALF)
                dump = (d16 & 127) + (N * (1 - cid))
                didx_v[slot, k, pl.ds(i * 16, 16)] = jnp.where(keep, d16, dump)

    def _drain_idx(slot):
        pltpu.make_async_copy(src_hbm.at[pl.ds(0, 8 * W)],
                              sidx_v.at[slot], isem).wait()
        pltpu.make_async_copy(dst2_hbm.at[pl.ds(0, 8)],
                              didx_v.at[slot], isem).wait()
        _remap_dst(slot)

    def _fire_g(B, slot, r):
        for k in range(4):
            pltpu.async_copy(
                a_sh.at[sidx_v.at[slot, pl.ds((r * 4 + k) * W, W)]],
                vals_v.at[4 * B + k], gsem.at[B])

    def _drain_g(B):
        for k in range(4):
            pltpu.make_async_copy(a_sh.at[pl.ds(0, W)],
                                  vals_v.at[4 * B + k], gsem.at[B]).wait()

    def _fire_s(B, slot, r):
        for k in range(4):
            pltpu.async_copy(vals_v.at[4 * B + k],
                             s2_sh.at[didx_v.at[slot, r * 4 + k]],
                             ssem.at[B], add=True)

    def _drain_s(B, slot, r):
        for k in range(4):
            pltpu.make_async_copy(vals_v.at[4 * B + k],
                                  s2_sh.at[didx_v.at[slot, r * 4 + k]],
                                  ssem.at[B]).wait()

    def _sync_idx0():
        pltpu.sync_copy(src_hbm.at[pl.ds(e0, 8 * W)], sidx_v.at[0])
        pltpu.sync_copy(dst2_hbm.at[pl.ds(w0, 8)], didx_v.at[0])
        _remap_dst(0)

    _pipeline(NCHUNK, _fire_idx, _drain_idx, _fire_g, _drain_g,
              _fire_s, _drain_s, _sync_idx0)
    plsc.subcore_barrier()

    rowf = lo + sid * FIN
    pltpu.sync_copy(s2_sh.at[pl.ds(rowf, FIN)], fs2_v)
    pltpu.sync_copy(cnt_hbm.at[pl.ds(rowf, FIN)], fcnt_v)
    pltpu.sync_copy(bc_hbm.at[pl.ds(rowf, FIN)], fbc_v)
    for i in range(FIN // 16):
        s2v = fs2_v[pl.ds(i * 16, 16)]
        cntv = jnp.maximum(fcnt_v[pl.ds(i * 16, 16)], 1.0)
        bcv = fbc_v[pl.ds(i * 16, 16)]
        fout_v[pl.ds(i * 16, 16)] = s2v / cntv + bcv
    pltpu.sync_copy(fout_v, out_hbm.at[pl.ds(rowf, FIN)])


@functools.cache
def _sc_kernels():
    mesh = plsc.VectorSubcoreMesh(core_axis_name="c", subcore_axis_name="s")
    sc_edge = pl.kernel(
        _sc_edge,
        mesh=mesh,
        compiler_params=pltpu.CompilerParams(use_tc_tiling_on_sc=False),
        out_type=[
            jax.ShapeDtypeStruct((2 * NPAD, FH), jnp.float32),
            jax.ShapeDtypeStruct((NPAD,), jnp.float32),
        ],
        scratch_types=[
            pltpu.VMEM((2, 8 * W), jnp.int32),
            pltpu.VMEM((2, 8, W), jnp.int32),
            pltpu.VMEM((8, W, FH), jnp.float32),
            pltpu.VMEM((W,), jnp.float32),
            pltpu.VMEM((32, FH), jnp.float32),
            pltpu.VMEM((ROWS_PER_TILE,), jnp.float32),
            pltpu.VMEM_SHARED((NPAD, FH), jnp.float32),
            pltpu.VMEM_SHARED((NPAD,), jnp.float32),
            pltpu.SemaphoreType.DMA((2,)),
            pltpu.SemaphoreType.DMA((2,)),
            pltpu.SemaphoreType.DMA,
            pltpu.SemaphoreType.DMA,
        ],
    )
    sc_scalar = pl.kernel(
        _sc_scalar,
        mesh=mesh,
        out_type=jax.ShapeDtypeStruct((NPAD,), jnp.float32),
        scratch_types=[
            pltpu.VMEM((2, 8 * W), jnp.int32),
            pltpu.VMEM((2, 8, W), jnp.int32),
            pltpu.VMEM((8, W), jnp.float32),
            pltpu.VMEM((ROWS_PER_TILE,), jnp.float32),
            pltpu.VMEM((FIN,), jnp.float32),
            pltpu.VMEM((FIN,), jnp.float32),
            pltpu.VMEM((FIN,), jnp.float32),
            pltpu.VMEM((FIN,), jnp.float32),
            pltpu.VMEM_SHARED((NPAD,), jnp.float32),
            pltpu.VMEM_SHARED((NPAD,), jnp.float32),
            pltpu.SemaphoreType.DMA((2,)),
            pltpu.SemaphoreType.DMA((2,)),
            pltpu.SemaphoreType.DMA,
        ],
    )
    return sc_edge, sc_scalar


def _tc1_body(s1a, s1b, cnt, xb, wl1ta, wl1tb, wr1t, wl2t, wr2t, wlin_t,
              bl1, bl2_col, blin_c, a_out, bc_out):
    cnt_c = jnp.maximum(cnt[...], 1.0)
    h1 = jnp.maximum(
        jnp.dot(s1a[...] / cnt_c, wl1ta[...],
                preferred_element_type=jnp.float32)
        + jnp.dot(s1b[...] / cnt_c, wl1tb[...],
                  preferred_element_type=jnp.float32)
        + bl1[...]
        + jnp.dot(xb[...], wr1t[...], preferred_element_type=jnp.float32),
        0.0)
    u_col = jnp.dot(wl2t[...], wlin_t[...], preferred_element_type=jnp.float32)
    w_col = jnp.dot(wr2t[...], wlin_t[...], preferred_element_type=jnp.float32)
    c = (jnp.dot(bl2_col[...].T, wlin_t[...],
                 preferred_element_type=jnp.float32) + blin_c[...])
    a_out[...] = jnp.dot(h1, u_col, preferred_element_type=jnp.float32)
    bc_out[...] = jnp.dot(h1, w_col, preferred_element_type=jnp.float32) + c


_tc1 = pl.pallas_call(
    _tc1_body,
    grid=(GRID,),
    in_specs=[
        pl.BlockSpec((R, FH), lambda i: (i, 0)),
        pl.BlockSpec((R, FH), lambda i: (i, 0)),
        pl.BlockSpec((R, 1), lambda i: (i, 0)),
        pl.BlockSpec((R, F), lambda i: (i, 0)),
        pl.BlockSpec((FH, F), lambda i: (0, 0)),
        pl.BlockSpec((FH, F), lambda i: (0, 0)),
        pl.BlockSpec((F, F), lambda i: (0, 0)),
        pl.BlockSpec((F, F), lambda i: (0, 0)),
        pl.BlockSpec((F, F), lambda i: (0, 0)),
        pl.BlockSpec((F, 1), lambda i: (0, 0)),
        pl.BlockSpec((1, F), lambda i: (0, 0)),
        pl.BlockSpec((F, 1), lambda i: (0, 0)),
        pl.BlockSpec((1, 1), lambda i: (0, 0)),
    ],
    out_specs=[
        pl.BlockSpec((R, 1), lambda i: (i, 0)),
        pl.BlockSpec((R, 1), lambda i: (i, 0)),
    ],
    out_shape=[
        jax.ShapeDtypeStruct((N, 1), jnp.float32),
        jax.ShapeDtypeStruct((N, 1), jnp.float32),
    ],
)


def kernel(x, edge_index, Wl1, bl1, Wr1, Wl2, bl2, Wr2, Wlin, blin):
    sc_edge, sc_scalar = _sc_kernels()

    npad_e = EPAD - E
    pad_src = (jnp.arange(npad_e, dtype=jnp.int32) * 37) % N
    pad_dst = N + (jnp.arange(npad_e, dtype=jnp.int32) % (NPAD - N))
    srcp = jnp.concatenate([edge_index[0], pad_src])
    dstp = jnp.concatenate([edge_index[1], pad_dst])
    dst2 = dstp.reshape(EPAD // W, W)
    x2 = jnp.concatenate([x[:, :FH], x[:, FH:]], axis=0)

    s1_flat, cnt_flat = sc_edge(x2, srcp, dst2)
    s1a, s1b = s1_flat[:N], s1_flat[NPAD:NPAD + N]
    cnt_col = cnt_flat[:N, None]

    a_col, bc_col = _tc1(s1a, s1b, cnt_col, x,
                         Wl1.T[:FH], Wl1.T[FH:], Wr1.T, Wl2.T, Wr2.T, Wlin.T,
                         bl1[None, :], bl2[:, None], blin[:, None])
    a_pad = jnp.pad(a_col[:, 0], (0, NPAD - N))
    bc_pad = jnp.pad(bc_col[:, 0], (0, NPAD - N))

    out_pad = sc_scalar(a_pad, srcp, dst2, cnt_flat, bc_pad)
    return out_pad[:N]
